# baseline (speedup 1.0000x reference)
import numpy as np
import jax
import jax.numpy as jnp
from jax.experimental import pallas as pl
from jax.experimental.pallas import tpu as pltpu

H = W = 64
HW = H * W
NM = 9

_INTERPRET = False

_TAPS = []
_VMASK_NP = np.zeros((9, HW), dtype=np.float32)
for _kh in range(3):
    for _kw in range(3):
        _dy, _dx = _kh - 1, _kw - 1
        _delta = _dy * W + _dx
        _hh, _ww = np.meshgrid(np.arange(H), np.arange(W), indexing="ij")
        _valid = ((_hh + _dy >= 0) & (_hh + _dy < H)
                  & (_ww + _dx >= 0) & (_ww + _dx < W))
        _k = _kh * 3 + _kw
        _VMASK_NP[_k] = _valid.reshape(-1).astype(np.float32)
        _TAPS.append(((-_delta) % HW, _k))

_PBIG_NP = np.zeros((NM * 32, NM * 32), dtype=np.float32)
for _a in range(32):
    for _i in range(NM):
        _PBIG_NP[_a * NM + _i, _i * 32 + _a] = 1.0

_E0T_NP = np.zeros((32, 1024), dtype=np.float32)
for _p in range(32):
    _E0T_NP[_p, _p * 32] = 1.0

_ROW_B = 0
_ROW_FCB = 8
_ROW_COMPW = 16
_ROW_UNZWT = 48
_ROW_PBIG = 80
_ROW_E0T = 368
_SP_ROWS = 400


def _last_valid_onehot(cond):
    notafter = jnp.ones((1, HW), dtype=jnp.float32)
    rows = [None] * NM
    for i in range(NM - 1, -1, -1):
        ci = cond[i:i + 1, :]
        rows[i] = ci * notafter
        notafter = notafter * (1.0 - ci)
    return jnp.concatenate(rows, axis=0)


def _dot(a, b):
    return jax.lax.dot_general(a, b, (((1,), (0,)), ((), ())),
                               preferred_element_type=jnp.float32)


def _dot_t(a, b):
    return jax.lax.dot_general(a, b, (((1,), (1,)), ((), ())),
                               preferred_element_type=jnp.float32)


def _dot_c0(a, b):
    return jax.lax.dot_general(a, b, (((0,), (0,)), ((), ())),
                               preferred_element_type=jnp.float32)


def _conv3x3(h, wbuf, cout, b, vm, relu=True):
    cin = h.shape[0]
    xcat = jnp.concatenate(
        [jnp.roll(h, 1, axis=1) * vm[3:4, :], h,
         jnp.roll(h, HW - 1, axis=1) * vm[5:6, :]], axis=0)
    acc = None
    for dy in (-1, 0, 1):
        z = _dot(wbuf[(dy + 1) * cout:(dy + 2) * cout, 0:3 * cin], xcat)
        if dy == -1:
            z = jnp.roll(z, 64, axis=1) * vm[1:2, :]
        elif dy == 1:
            z = jnp.roll(z, HW - 64, axis=1) * vm[7:8, :]
        acc = z if acc is None else acc + z
    acc = acc + b
    return jnp.maximum(acc, 0.0) if relu else acc


def _branch(x, m, vm, w1_ref, b1, w2_ref, b2, w3_ref, b3, pbig, bbuf,
            wait_w):
    cnt = jnp.sum(m, axis=1, keepdims=True)
    inv = 1.0 / jnp.maximum(cnt, 1.0)
    ok = (cnt >= 10.0).astype(jnp.float32)

    sums = _dot_t(x, m)
    cond = m * ok
    S = _last_valid_onehot(cond)
    fsm = x - _dot(sums, S * inv)

    wait_w(0)
    h1 = _conv3x3(fsm, w1_ref, 128, b1, vm)
    wait_w(1)
    h2 = _conv3x3(h1, w2_ref, 64, b2, vm)
    wait_w(2)
    h3 = _conv3x3(h2, w3_ref, 32, b3, vm, relu=False)

    minv = m * inv
    for i in range(NM):
        bbuf[i * 32:(i + 1) * 32, :] = h3 * minv[i:i + 1, :]
    covs = _dot_t(bbuf[...], h3)
    amaj = _dot(pbig, covs)
    covs_flat = jnp.concatenate(
        [amaj[a * NM:(a + 1) * NM, :] for a in range(32)], axis=1)
    return covs_flat, fsm, cnt


def _col(row, n):
    return jnp.transpose(row[:, 0:n], (1, 0))


def _mega_body(mv_ref, sp_ref,
               sx_hbm, cx_hbm, wpack_hbm, sfcw_hbm, cfcw_hbm,
               out_ref,
               sx_v, cx_v, sw1_v, sw2_v, sw3_v, cw1_v, cw2_v, cw3_v,
               sfcw_v, cfcw_v, bbuf,
               *sems):
    copies = [
        pltpu.make_async_copy(sx_hbm, sx_v, sems[0]),
        pltpu.make_async_copy(wpack_hbm.at[0:384, :], sw1_v, sems[1]),
        pltpu.make_async_copy(cx_hbm, cx_v, sems[2]),
        pltpu.make_async_copy(wpack_hbm.at[672 + 0:672 + 384, :],
                              cw1_v, sems[3]),
        pltpu.make_async_copy(wpack_hbm.at[384:576, 0:384], sw2_v, sems[4]),
        pltpu.make_async_copy(wpack_hbm.at[576:672, 0:256], sw3_v, sems[5]),
        pltpu.make_async_copy(wpack_hbm.at[672 + 384:672 + 576, 0:384],
                              cw2_v, sems[6]),
        pltpu.make_async_copy(wpack_hbm.at[672 + 576:672 + 672, 0:256],
                              cw3_v, sems[7]),
        pltpu.make_async_copy(sfcw_hbm, sfcw_v, sems[8]),
        pltpu.make_async_copy(cfcw_hbm, cfcw_v, sems[9]),
    ]
    for cp in copies:
        cp.start()

    cm = mv_ref[0:NM, :]
    sm = mv_ref[NM:2 * NM, :]
    vm = mv_ref[2 * NM:3 * NM, :]

    cb1 = _col(sp_ref[_ROW_B + 0:_ROW_B + 1, :], 128)
    cb2 = _col(sp_ref[_ROW_B + 1:_ROW_B + 2, :], 64)
    cb3 = _col(sp_ref[_ROW_B + 2:_ROW_B + 3, :], 32)
    sb1 = _col(sp_ref[_ROW_B + 3:_ROW_B + 4, :], 128)
    sb2 = _col(sp_ref[_ROW_B + 4:_ROW_B + 5, :], 64)
    sb3 = _col(sp_ref[_ROW_B + 5:_ROW_B + 6, :], 32)
    cfcb = sp_ref[_ROW_FCB + 0:_ROW_FCB + 1, :]
    sfcb = sp_ref[_ROW_FCB + 1:_ROW_FCB + 2, :]
    compb = _col(sp_ref[_ROW_FCB + 2:_ROW_FCB + 3, :], 32)
    unzb = _col(sp_ref[_ROW_FCB + 3:_ROW_FCB + 4, :], 256)
    compw = sp_ref[_ROW_COMPW:_ROW_COMPW + 32, 0:256]
    unzwT = sp_ref[_ROW_UNZWT:_ROW_UNZWT + 32, 0:256]
    pbig = sp_ref[_ROW_PBIG:_ROW_PBIG + 288, 0:288]
    e0t = sp_ref[_ROW_E0T:_ROW_E0T + 32, :]

    copies[0].wait()
    sx = sx_v[...]

    def wait_sw(j):
        copies[[1, 4, 5][j]].wait()

    scovs, _, scnt = _branch(sx, sm, vm, sw1_v, sb1, sw2_v, sb2,
                             sw3_v, sb3, pbig, bbuf, wait_sw)
    ssums = _dot_t(sx, sm)

    copies[2].wait()
    cx = cx_v[...]

    def wait_cw(j):
        copies[[3, 6, 7][j]].wait()

    ccovs, cfsm, ccnt = _branch(cx, cm, vm, cw1_v, cb1, cw2_v, cb2,
                                cw3_v, cb3, pbig, bbuf, wait_cw)

    copies[8].wait()
    sM = _dot_t(scovs, sfcw_v[...]) + sfcb
    copies[9].wait()
    cM = _dot_t(ccovs, cfcw_v[...]) + cfcb

    ccf = _dot(compw, cfsm) + compb

    sinv = 1.0 / jnp.maximum(scnt, 1.0)

    valid = ((ccnt >= 10.0) & (scnt >= 10.0)).astype(jnp.float32)
    cond = cm * valid
    S = _last_valid_onehot(cond)
    anyS = jnp.sum(S, axis=0, keepdims=True)

    acc = ccf * (1.0 - anyS)
    for r in range(32):
        G = _dot(cM[:, r * 32:(r + 1) * 32], ccf)
        rolled = sM if r == 0 else jnp.roll(sM, -r, axis=1)
        A = _dot_t(rolled, e0t)
        acc = acc + _dot_c0(A, S * G)
    fsmean = _dot(ssums, S * sinv)
    out_ref[...] = _dot_c0(unzwT, acc) + unzb + fsmean


def kernel(cF, sF, cmasks, smasks, s_c1w, s_c1b, s_c2w, s_c2b, s_c3w, s_c3b,
           s_fcw, s_fcb, c_c1w, c_c1b, c_c2w, c_c2b, c_c3w, c_c3b, c_fcw,
           c_fcb, comp_w, comp_b, unzip_w, unzip_b):
    f32 = jnp.float32
    cmf = (cmasks[:, 0].reshape(NM, HW) == 1).astype(f32)
    smf = (smasks[:, 0].reshape(NM, HW) == 1).astype(f32)
    mvpack = jnp.concatenate([cmf, smf, jnp.asarray(_VMASK_NP)], axis=0)

    def sect(a, rows):
        return jnp.pad(a, ((0, rows - a.shape[0]), (0, 1024 - a.shape[1])))

    biases = jnp.stack([
        jnp.pad(c_c1b, (0, 1024 - 128)), jnp.pad(c_c2b, (0, 1024 - 64)),
        jnp.pad(c_c3b, (0, 1024 - 32)),
        jnp.pad(s_c1b, (0, 1024 - 128)), jnp.pad(s_c2b, (0, 1024 - 64)),
        jnp.pad(s_c3b, (0, 1024 - 32))])
    fcbs = jnp.stack([c_fcb, s_fcb, jnp.pad(comp_b, (0, 1024 - 32)),
                      jnp.pad(unzip_b, (0, 1024 - 256))])
    smallpack = jnp.concatenate([
        sect(biases, 8),
        sect(fcbs, 8),
        sect(comp_w.reshape(32, 256), 32),
        sect(jnp.transpose(unzip_w.reshape(256, 32), (1, 0)), 32),
        sect(jnp.asarray(_PBIG_NP), 288),
        jnp.asarray(_E0T_NP)], axis=0)

    def taps(w, padto):
        t = jnp.transpose(w, (2, 0, 3, 1)).reshape(3 * w.shape[0],
                                                   3 * w.shape[1])
        return jnp.pad(t, ((0, 0), (0, padto - t.shape[1])))

    wpack = jnp.concatenate(
        [taps(s_c1w, 768), taps(s_c2w, 768), taps(s_c3w, 768),
         taps(c_c1w, 768), taps(c_c2w, 768), taps(c_c3w, 768)], axis=0)

    vspec = pl.BlockSpec(memory_space=pltpu.MemorySpace.VMEM)
    hspec = pl.BlockSpec(memory_space=pltpu.MemorySpace.HBM)
    vmem = pltpu.VMEM

    out = pl.pallas_call(
        _mega_body,
        out_shape=jax.ShapeDtypeStruct((256, HW), f32),
        in_specs=[vspec] * 2 + [hspec] * 5,
        out_specs=vspec,
        scratch_shapes=[
            vmem((256, HW), f32), vmem((256, HW), f32),
            vmem((384, 768), f32), vmem((192, 384), f32),
            vmem((96, 256), f32),
            vmem((384, 768), f32), vmem((192, 384), f32),
            vmem((96, 256), f32),
            vmem((1024, 1024), f32), vmem((1024, 1024), f32),
            vmem((NM * 32, HW), f32),
        ] + [pltpu.SemaphoreType.DMA] * 10,
        interpret=_INTERPRET,
    )(mvpack, smallpack, sF.reshape(256, HW), cF.reshape(256, HW),
      wpack, s_fcw, c_fcw)

    return out.reshape(1, 256, H, W)


# speedup vs baseline: 1.0208x; 1.0208x over previous
import numpy as np
import jax
import jax.numpy as jnp
from jax.experimental import pallas as pl
from jax.experimental.pallas import tpu as pltpu

H = W = 64
HW = H * W
NM = 9

_INTERPRET = False

_TAPS = []
_VMASK_NP = np.zeros((9, HW), dtype=np.float32)
for _kh in range(3):
    for _kw in range(3):
        _dy, _dx = _kh - 1, _kw - 1
        _delta = _dy * W + _dx
        _hh, _ww = np.meshgrid(np.arange(H), np.arange(W), indexing="ij")
        _valid = ((_hh + _dy >= 0) & (_hh + _dy < H)
                  & (_ww + _dx >= 0) & (_ww + _dx < W))
        _k = _kh * 3 + _kw
        _VMASK_NP[_k] = _valid.reshape(-1).astype(np.float32)
        _TAPS.append(((-_delta) % HW, _k))

_PBIG_NP = np.zeros((NM * 32, NM * 32), dtype=np.float32)
for _a in range(32):
    for _i in range(NM):
        _PBIG_NP[_a * NM + _i, _i * 32 + _a] = 1.0

_E0T_NP = np.zeros((32, 1024), dtype=np.float32)
for _p in range(32):
    _E0T_NP[_p, _p * 32] = 1.0

_ROW_B = 0
_ROW_FCB = 8
_ROW_COMPW = 16
_ROW_UNZWT = 48
_ROW_PBIG = 80
_ROW_E0T = 368
_SP_ROWS = 400


def _last_valid_onehot(cond):
    notafter = jnp.ones((1, HW), dtype=jnp.float32)
    rows = [None] * NM
    for i in range(NM - 1, -1, -1):
        ci = cond[i:i + 1, :]
        rows[i] = ci * notafter
        notafter = notafter * (1.0 - ci)
    return jnp.concatenate(rows, axis=0)


def _dot(a, b):
    return jax.lax.dot_general(a, b, (((1,), (0,)), ((), ())),
                               preferred_element_type=jnp.float32)


def _dot_t(a, b):
    return jax.lax.dot_general(a, b, (((1,), (1,)), ((), ())),
                               preferred_element_type=jnp.float32)


def _dot_c0(a, b):
    return jax.lax.dot_general(a, b, (((0,), (0,)), ((), ())),
                               preferred_element_type=jnp.float32)


def _conv3x3(h, wbuf, cout, b, vm, relu=True):
    cin = h.shape[0]
    xs = [jnp.roll(h, 1, axis=1) * vm[3:4, :], h,
          jnp.roll(h, HW - 1, axis=1) * vm[5:6, :]]
    acc = None
    for dy in (-1, 0, 1):
        z = None
        for dx in (-1, 0, 1):
            k = (dy + 1) * 3 + (dx + 1)
            t = _dot(wbuf[k * cout:(k + 1) * cout, 0:cin], xs[dx + 1])
            z = t if z is None else z + t
        if dy == -1:
            z = jnp.roll(z, 64, axis=1) * vm[1:2, :]
        elif dy == 1:
            z = jnp.roll(z, HW - 64, axis=1) * vm[7:8, :]
        acc = z if acc is None else acc + z
    acc = acc + b
    return jnp.maximum(acc, 0.0) if relu else acc


def _branch(x, m, vm, w1_ref, b1, w2_ref, b2, w3_ref, b3, pbig, bbuf,
            wait_w):
    cnt = jnp.sum(m, axis=1, keepdims=True)
    inv = 1.0 / jnp.maximum(cnt, 1.0)
    ok = (cnt >= 10.0).astype(jnp.float32)

    sums = _dot_t(x, m)
    cond = m * ok
    S = _last_valid_onehot(cond)
    fsm = x - _dot(sums, S * inv)

    wait_w(0)
    h1 = _conv3x3(fsm, w1_ref, 128, b1, vm)
    wait_w(1)
    h2 = _conv3x3(h1, w2_ref, 64, b2, vm)
    wait_w(2)
    h3 = _conv3x3(h2, w3_ref, 32, b3, vm, relu=False)

    minv = m * inv
    for i in range(NM):
        bbuf[i * 32:(i + 1) * 32, :] = h3 * minv[i:i + 1, :]
    covs = _dot_t(bbuf[...], h3)
    amaj = _dot(pbig, covs)
    covs_flat = jnp.concatenate(
        [amaj[a * NM:(a + 1) * NM, :] for a in range(32)], axis=1)
    return covs_flat, fsm, cnt


def _col(row, n):
    return jnp.transpose(row[:, 0:n], (1, 0))


def _mega_body(mv_ref, sp_ref,
               sx_hbm, cx_hbm, wpack_hbm, sfcw_hbm, cfcw_hbm,
               out_ref,
               sx_v, cx_v, sw1_v, sw2_v, sw3_v, cw1_v, cw2_v, cw3_v,
               sfcw_v, cfcw_v, bbuf,
               *sems):
    copies = [
        pltpu.make_async_copy(sx_hbm, sx_v, sems[0]),
        pltpu.make_async_copy(wpack_hbm.at[0:1152, :], sw1_v, sems[1]),
        pltpu.make_async_copy(cx_hbm, cx_v, sems[2]),
        pltpu.make_async_copy(wpack_hbm.at[2016 + 0:2016 + 1152, :],
                              cw1_v, sems[3]),
        pltpu.make_async_copy(wpack_hbm.at[1152:1728, 0:128], sw2_v, sems[4]),
        pltpu.make_async_copy(wpack_hbm.at[1728:2016, 0:128], sw3_v, sems[5]),
        pltpu.make_async_copy(wpack_hbm.at[2016 + 1152:2016 + 1728, 0:128],
                              cw2_v, sems[6]),
        pltpu.make_async_copy(wpack_hbm.at[2016 + 1728:2016 + 2016, 0:128],
                              cw3_v, sems[7]),
        pltpu.make_async_copy(sfcw_hbm, sfcw_v, sems[8]),
        pltpu.make_async_copy(cfcw_hbm, cfcw_v, sems[9]),
    ]
    for cp in copies:
        cp.start()

    cm = mv_ref[0:NM, :]
    sm = mv_ref[NM:2 * NM, :]
    vm = mv_ref[2 * NM:3 * NM, :]

    cb1 = _col(sp_ref[_ROW_B + 0:_ROW_B + 1, :], 128)
    cb2 = _col(sp_ref[_ROW_B + 1:_ROW_B + 2, :], 64)
    cb3 = _col(sp_ref[_ROW_B + 2:_ROW_B + 3, :], 32)
    sb1 = _col(sp_ref[_ROW_B + 3:_ROW_B + 4, :], 128)
    sb2 = _col(sp_ref[_ROW_B + 4:_ROW_B + 5, :], 64)
    sb3 = _col(sp_ref[_ROW_B + 5:_ROW_B + 6, :], 32)
    cfcb = sp_ref[_ROW_FCB + 0:_ROW_FCB + 1, :]
    sfcb = sp_ref[_ROW_FCB + 1:_ROW_FCB + 2, :]
    compb = _col(sp_ref[_ROW_FCB + 2:_ROW_FCB + 3, :], 32)
    unzb = _col(sp_ref[_ROW_FCB + 3:_ROW_FCB + 4, :], 256)
    compw = sp_ref[_ROW_COMPW:_ROW_COMPW + 32, 0:256]
    unzwT = sp_ref[_ROW_UNZWT:_ROW_UNZWT + 32, 0:256]
    pbig = sp_ref[_ROW_PBIG:_ROW_PBIG + 288, 0:288]
    e0t = sp_ref[_ROW_E0T:_ROW_E0T + 32, :]

    copies[0].wait()
    sx = sx_v[...]

    def wait_sw(j):
        copies[[1, 4, 5][j]].wait()

    scovs, _, scnt = _branch(sx, sm, vm, sw1_v, sb1, sw2_v, sb2,
                             sw3_v, sb3, pbig, bbuf, wait_sw)
    ssums = _dot_t(sx, sm)

    copies[2].wait()
    cx = cx_v[...]

    def wait_cw(j):
        copies[[3, 6, 7][j]].wait()

    ccovs, cfsm, ccnt = _branch(cx, cm, vm, cw1_v, cb1, cw2_v, cb2,
                                cw3_v, cb3, pbig, bbuf, wait_cw)

    copies[8].wait()
    sM = _dot_t(scovs, sfcw_v[...]) + sfcb
    copies[9].wait()
    cM = _dot_t(ccovs, cfcw_v[...]) + cfcb

    ccf = _dot(compw, cfsm) + compb

    sinv = 1.0 / jnp.maximum(scnt, 1.0)

    valid = ((ccnt >= 10.0) & (scnt >= 10.0)).astype(jnp.float32)
    cond = cm * valid
    S = _last_valid_onehot(cond)
    anyS = jnp.sum(S, axis=0, keepdims=True)

    acc = ccf * (1.0 - anyS)
    for r in range(32):
        G = _dot(cM[:, r * 32:(r + 1) * 32], ccf)
        rolled = sM if r == 0 else jnp.roll(sM, -r, axis=1)
        A = _dot_t(rolled, e0t)
        acc = acc + _dot_c0(A, S * G)
    fsmean = _dot(ssums, S * sinv)
    out_ref[...] = _dot_c0(unzwT, acc) + unzb + fsmean


def kernel(cF, sF, cmasks, smasks, s_c1w, s_c1b, s_c2w, s_c2b, s_c3w, s_c3b,
           s_fcw, s_fcb, c_c1w, c_c1b, c_c2w, c_c2b, c_c3w, c_c3b, c_fcw,
           c_fcb, comp_w, comp_b, unzip_w, unzip_b):
    f32 = jnp.float32
    cmf = (cmasks[:, 0].reshape(NM, HW) == 1).astype(f32)
    smf = (smasks[:, 0].reshape(NM, HW) == 1).astype(f32)
    mvpack = jnp.concatenate([cmf, smf, jnp.asarray(_VMASK_NP)], axis=0)

    def sect(a, rows):
        return jnp.pad(a, ((0, rows - a.shape[0]), (0, 1024 - a.shape[1])))

    biases = jnp.stack([
        jnp.pad(c_c1b, (0, 1024 - 128)), jnp.pad(c_c2b, (0, 1024 - 64)),
        jnp.pad(c_c3b, (0, 1024 - 32)),
        jnp.pad(s_c1b, (0, 1024 - 128)), jnp.pad(s_c2b, (0, 1024 - 64)),
        jnp.pad(s_c3b, (0, 1024 - 32))])
    fcbs = jnp.stack([c_fcb, s_fcb, jnp.pad(comp_b, (0, 1024 - 32)),
                      jnp.pad(unzip_b, (0, 1024 - 256))])
    smallpack = jnp.concatenate([
        sect(biases, 8),
        sect(fcbs, 8),
        sect(comp_w.reshape(32, 256), 32),
        sect(jnp.transpose(unzip_w.reshape(256, 32), (1, 0)), 32),
        sect(jnp.asarray(_PBIG_NP), 288),
        jnp.asarray(_E0T_NP)], axis=0)

    def taps(w, padto):
        t = jnp.transpose(w, (2, 3, 0, 1)).reshape(9 * w.shape[0], w.shape[1])
        return jnp.pad(t, ((0, 0), (0, padto - t.shape[1])))

    wpack = jnp.concatenate(
        [taps(s_c1w, 256), taps(s_c2w, 256), taps(s_c3w, 256),
         taps(c_c1w, 256), taps(c_c2w, 256), taps(c_c3w, 256)], axis=0)

    vspec = pl.BlockSpec(memory_space=pltpu.MemorySpace.VMEM)
    hspec = pl.BlockSpec(memory_space=pltpu.MemorySpace.HBM)
    vmem = pltpu.VMEM

    out = pl.pallas_call(
        _mega_body,
        out_shape=jax.ShapeDtypeStruct((256, HW), f32),
        in_specs=[vspec] * 2 + [hspec] * 5,
        out_specs=vspec,
        scratch_shapes=[
            vmem((256, HW), f32), vmem((256, HW), f32),
            vmem((1152, 256), f32), vmem((576, 128), f32),
            vmem((288, 128), f32),
            vmem((1152, 256), f32), vmem((576, 128), f32),
            vmem((288, 128), f32),
            vmem((1024, 1024), f32), vmem((1024, 1024), f32),
            vmem((NM * 32, HW), f32),
        ] + [pltpu.SemaphoreType.DMA] * 10,
        interpret=_INTERPRET,
    )(mvpack, smallpack, sF.reshape(256, HW), cF.reshape(256, HW),
      wpack, s_fcw, c_fcw)

    return out.reshape(1, 256, H, W)
